# baseline (speedup 1.0000x reference)
import functools

import jax
import jax.numpy as jnp
from jax import lax
from jax.experimental import pallas as pl
from jax.experimental.pallas import tpu as pltpu
from jax.experimental.pallas import tpu_sc as plsc

N_NODES = 16384
SUBTOK = 20
EMBED = 128
L = 16
NF = EMBED // L
NC, NS = 2, 16
NW = NC * NS
NPW = N_NODES // NW
C = 16
NCHUNK = NPW // C


def _sc_embed_sum(x_flat, table):
    mesh = plsc.VectorSubcoreMesh(core_axis_name="c", subcore_axis_name="s")

    @functools.partial(
        pl.kernel,
        mesh=mesh,
        out_type=jax.ShapeDtypeStruct((N_NODES, EMBED), jnp.float32),
        compiler_params=pltpu.CompilerParams(needs_layout_passes=False),
        scratch_types=[
            pltpu.VMEM((NPW * SUBTOK,), jnp.int32),
            pltpu.VMEM((C * SUBTOK, EMBED), jnp.float32),
            pltpu.VMEM((C * SUBTOK, EMBED), jnp.float32),
            pltpu.VMEM((C, EMBED), jnp.float32),
            pltpu.VMEM((C, EMBED), jnp.float32),
            pltpu.VMEM((1, EMBED), jnp.float32),
            pltpu.VMEM((L,), jnp.float32),
            pltpu.SemaphoreType.DMA,
            pltpu.SemaphoreType.DMA,
            pltpu.SemaphoreType.DMA,
            pltpu.SemaphoreType.DMA,
        ],
    )
    def k(x_hbm, table_hbm, out_hbm, idx_v, rows0_v, rows1_v,
          outb0_v, outb1_v, t0_v, cnt_v, semg0, semg1, semo0, semo1):
        rows_b = (rows0_v, rows1_v)
        outb_b = (outb0_v, outb1_v)
        semg_b = (semg0, semg1)
        semo_b = (semo0, semo1)
        wid = lax.axis_index("s") * NC + lax.axis_index("c")
        nbase = wid * NPW

        def gather_copy(g, b):
            return pltpu.make_async_copy(
                table_hbm.at[idx_v.at[pl.ds(g * C * SUBTOK, C * SUBTOK)]],
                rows_b[b], semg_b[b])

        def out_copy(g, b):
            nb = nbase + g * C
            return pltpu.make_async_copy(
                outb_b[b], out_hbm.at[pl.ds(nb, C)], semo_b[b])

        pltpu.sync_copy(
            x_hbm.at[pl.ds(nbase * SUBTOK, NPW * SUBTOK)], idx_v)
        pltpu.sync_copy(table_hbm.at[pl.ds(0, 1)], t0_v)
        gather_copy(0, 0).start()

        def pair_body(j, carry):
            for p in range(2):
                g = 2 * j + p
                bg = p
                bn = 1 - p

                @pl.when(g + 1 < NCHUNK)
                def _():
                    gather_copy(g + 1, bn).start()

                lanes = lax.iota(jnp.int32, L)
                node_base = lanes * SUBTOK + g * (C * SUBTOK)
                one = jnp.full((L,), 1.0, jnp.float32)
                zero = jnp.zeros((L,), jnp.float32)
                cntv = zero
                for s in range(SUBTOK):
                    vals = plsc.load_gather(idx_v, [node_base + s])
                    cntv = cntv + jnp.where(vals == 0, one, zero)
                cnt_v[...] = cntv

                gather_copy(g, bg).wait()

                @pl.when(g >= 2)
                def _():
                    out_copy(g, bg).wait()

                rows_v = rows_b[bg]
                outb_v = outb_b[bg]

                def node_body(n, carry2):
                    r0 = n * SUBTOK
                    cfv = cnt_v[...].at[jnp.full((L,), n, jnp.int32)].get(
                        mode="promise_in_bounds")
                    for f0 in range(0, NF, 4):
                        acc = [rows_v[r0, pl.ds((f0 + f) * L, L)]
                               for f in range(4)]
                        for s in range(1, SUBTOK):
                            for f in range(4):
                                acc[f] = acc[f] + rows_v[
                                    r0 + s, pl.ds((f0 + f) * L, L)]
                        for f in range(4):
                            outb_v[n, pl.ds((f0 + f) * L, L)] = (
                                acc[f] - cfv * t0_v[0, pl.ds((f0 + f) * L, L)])
                    return carry2

                lax.fori_loop(0, C, node_body, 0)
                out_copy(g, bg).start()
            return carry

        lax.fori_loop(0, NCHUNK // 2, pair_body, 0)
        out_copy(NCHUNK - 2, 0).wait()
        out_copy(NCHUNK - 1, 1).wait()

    return k(x_flat, table)


def kernel(x, table):
    x = x.astype(jnp.int32)
    return _sc_embed_sum(x.reshape(-1), table)


# speedup vs baseline: 1.0105x; 1.0105x over previous
import functools

import jax
import jax.numpy as jnp
from jax import lax
from jax.experimental import pallas as pl
from jax.experimental.pallas import tpu as pltpu
from jax.experimental.pallas import tpu_sc as plsc

N_NODES = 16384
SUBTOK = 20
EMBED = 128
L = 16
NF = EMBED // L
NC, NS = 2, 16
NW = NC * NS
NPW = N_NODES // NW
C = 16
NCHUNK = NPW // C


def _sc_embed_sum(x_flat, table):
    mesh = plsc.VectorSubcoreMesh(core_axis_name="c", subcore_axis_name="s")

    @functools.partial(
        pl.kernel,
        mesh=mesh,
        out_type=jax.ShapeDtypeStruct((N_NODES, EMBED), jnp.float32),
        compiler_params=pltpu.CompilerParams(needs_layout_passes=False),
        scratch_types=[
            pltpu.VMEM((C * SUBTOK,), jnp.int32),
            pltpu.VMEM((C * SUBTOK,), jnp.int32),
            pltpu.VMEM((C * SUBTOK, EMBED), jnp.float32),
            pltpu.VMEM((C * SUBTOK, EMBED), jnp.float32),
            pltpu.VMEM((C, EMBED), jnp.float32),
            pltpu.VMEM((C, EMBED), jnp.float32),
            pltpu.VMEM((1, EMBED), jnp.float32),
            pltpu.VMEM((L,), jnp.float32),
            pltpu.SemaphoreType.DMA,
            pltpu.SemaphoreType.DMA,
            pltpu.SemaphoreType.DMA,
            pltpu.SemaphoreType.DMA,
            pltpu.SemaphoreType.DMA,
            pltpu.SemaphoreType.DMA,
        ],
    )
    def k(x_hbm, table_hbm, out_hbm, idx0_v, idx1_v, rows0_v, rows1_v,
          outb0_v, outb1_v, t0_v, cnt_v, semi0, semi1, semg0, semg1,
          semo0, semo1):
        idx_b = (idx0_v, idx1_v)
        rows_b = (rows0_v, rows1_v)
        outb_b = (outb0_v, outb1_v)
        semi_b = (semi0, semi1)
        semg_b = (semg0, semg1)
        semo_b = (semo0, semo1)
        wid = lax.axis_index("s") * NC + lax.axis_index("c")
        nbase = wid * NPW
        pltpu.sync_copy(table_hbm.at[pl.ds(0, 1)], t0_v)

        def idx_copy(g, b):
            nb = nbase + g * C
            return pltpu.make_async_copy(
                x_hbm.at[pl.ds(nb * SUBTOK, C * SUBTOK)],
                idx_b[b], semi_b[b])

        def gather_copy(b):
            return pltpu.make_async_copy(
                table_hbm.at[idx_b[b]], rows_b[b], semg_b[b])

        def out_copy(g, b):
            nb = nbase + g * C
            return pltpu.make_async_copy(
                outb_b[b], out_hbm.at[pl.ds(nb, C)], semo_b[b])

        idx_copy(0, 0).start()
        idx_copy(0, 0).wait()
        gather_copy(0).start()
        idx_copy(1, 1).start()

        def pair_body(j, carry):
            for p in range(2):
                g = 2 * j + p
                bg = p
                bn = 1 - p

                @pl.when(g + 1 < NCHUNK)
                def _():
                    idx_copy(g + 1, bn).wait()
                    gather_copy(bn).start()

                rows_v = rows_b[bg]
                idx_v = idx_b[bg]

                lanes = lax.iota(jnp.int32, L)
                node_base = lanes * SUBTOK
                one = jnp.full((L,), 1.0, jnp.float32)
                zero = jnp.zeros((L,), jnp.float32)
                cntv = zero
                for s in range(SUBTOK):
                    vals = plsc.load_gather(idx_v, [node_base + s])
                    cntv = cntv + jnp.where(vals == 0, one, zero)
                cnt_v[...] = cntv

                gather_copy(bg).wait()

                @pl.when(g + 2 < NCHUNK)
                def _():
                    idx_copy(g + 2, bg).start()

                @pl.when(g >= 2)
                def _():
                    out_copy(g, bg).wait()

                outb_v = outb_b[bg]

                def node_body(n, carry2):
                    r0 = n * SUBTOK
                    cfv = cnt_v[...].at[jnp.full((L,), n, jnp.int32)].get(
                        mode="promise_in_bounds")
                    for f0 in range(0, NF, 4):
                        acc = [rows_v[r0, pl.ds((f0 + f) * L, L)]
                               for f in range(4)]
                        for s in range(1, SUBTOK):
                            for f in range(4):
                                acc[f] = acc[f] + rows_v[
                                    r0 + s, pl.ds((f0 + f) * L, L)]
                        for f in range(4):
                            outb_v[n, pl.ds((f0 + f) * L, L)] = (
                                acc[f] - cfv * t0_v[0, pl.ds((f0 + f) * L, L)])
                    return carry2

                lax.fori_loop(0, C, node_body, 0)
                out_copy(g, bg).start()
            return carry

        lax.fori_loop(0, NCHUNK // 2, pair_body, 0)
        out_copy(NCHUNK - 2, 0).wait()
        out_copy(NCHUNK - 1, 1).wait()

    return k(x_flat, table)


def kernel(x, table):
    x = x.astype(jnp.int32)
    return _sc_embed_sum(x.reshape(-1), table)
